# baseline (speedup 1.0000x reference)
import jax
import jax.numpy as jnp
from jax.experimental import pallas as pl
from jax.experimental.pallas import tpu as pltpu

N = 20000
BLK = 10240
GRID = (N + BLK - 1) // BLK


def _tc_body(rois_ref, gt_ref, lab_ref, tgt_ref, w_ref, w2_ref):
    x1 = rois_ref[1, 0, :]
    y1 = rois_ref[2, 0, :]
    x2 = rois_ref[3, 0, :]
    y2 = rois_ref[4, 0, :]
    gx1 = gt_ref[0, 0, :]
    gy1 = gt_ref[0, 1, :]
    gx2 = gt_ref[0, 2, :]
    gy2 = gt_ref[0, 3, :]
    lab = lab_ref[:]

    ew = x2 - x1 + 1.0
    eh = y2 - y1 + 1.0
    gw = gx2 - gx1 + 1.0
    gh = gy2 - gy1 + 1.0
    dx = ((gx1 + 0.5 * gw) - (x1 + 0.5 * ew)) / ew * 10.0
    dy = ((gy1 + 0.5 * gh) - (y1 + 0.5 * eh)) / eh * 10.0
    dw = jnp.log(gw / ew) * 5.0
    dh = jnp.log(gh / eh) * 5.0

    fg = lab > 0
    zero = jnp.zeros_like(dx)
    wv = jnp.where(fg, zero + 1.0, zero)
    tgt_ref[0, 0, :] = jnp.where(fg, dx, zero)
    tgt_ref[0, 1, :] = jnp.where(fg, dy, zero)
    tgt_ref[0, 2, :] = jnp.where(fg, dw, zero)
    tgt_ref[0, 3, :] = jnp.where(fg, dh, zero)
    w_ref[0, 0, :] = wv
    w_ref[0, 1, :] = wv
    w_ref[0, 2, :] = wv
    w_ref[0, 3, :] = wv
    w2_ref[0, 0, :] = wv
    w2_ref[0, 1, :] = wv
    w2_ref[0, 2, :] = wv
    w2_ref[0, 3, :] = wv


_out_bs = pl.BlockSpec((1, 4, BLK), lambda i: (0, 0, i))

_tc_call = pl.pallas_call(
    _tc_body,
    grid=(GRID,),
    in_specs=[
        pl.BlockSpec((5, 1, BLK), lambda i: (0, 0, i)),
        pl.BlockSpec((1, 4, BLK), lambda i: (0, 0, i)),
        pl.BlockSpec((BLK,), lambda i: (i,)),
    ],
    out_specs=[_out_bs, _out_bs, _out_bs],
    out_shape=(
        jax.ShapeDtypeStruct((1, 4, N), jnp.float32),
        jax.ShapeDtypeStruct((1, 4, N), jnp.float32),
        jax.ShapeDtypeStruct((1, 4, N), jnp.float32),
    ),
)


@jax.jit
def kernel(gt_rois, rois, labels):
    rois_t = jnp.transpose(rois, (2, 0, 1))
    gt_t = jnp.transpose(gt_rois, (0, 2, 1))
    rois_t = pltpu.with_memory_space_constraint(rois_t, pltpu.MemorySpace.HBM)
    gt_t = pltpu.with_memory_space_constraint(gt_t, pltpu.MemorySpace.HBM)
    tgt_t, w_t, w2_t = _tc_call(rois_t, gt_t, labels)
    tgt = jnp.transpose(tgt_t, (0, 2, 1))
    w = jnp.transpose(w_t, (0, 2, 1))
    w2 = jnp.transpose(w2_t, (0, 2, 1))
    return tgt, w, w2


# speedup vs baseline: 1.0074x; 1.0074x over previous
import jax
import jax.numpy as jnp
from jax.experimental import pallas as pl
from jax.experimental.pallas import tpu as pltpu

N = 20000
BLK = 10240
GRID = (N + BLK - 1) // BLK


def _tc_body(rois_ref, gt_ref, lab_ref, tgt_ref, w_ref, w2_ref):
    x1 = rois_ref[1, 0, :]
    y1 = rois_ref[2, 0, :]
    x2 = rois_ref[3, 0, :]
    y2 = rois_ref[4, 0, :]
    gx1 = gt_ref[0, 0, :]
    gy1 = gt_ref[0, 1, :]
    gx2 = gt_ref[0, 2, :]
    gy2 = gt_ref[0, 3, :]
    lab = lab_ref[:]

    ew = x2 - x1 + 1.0
    eh = y2 - y1 + 1.0
    gw = gx2 - gx1 + 1.0
    gh = gy2 - gy1 + 1.0
    dx = ((gx1 + 0.5 * gw) - (x1 + 0.5 * ew)) / ew * 10.0
    dy = ((gy1 + 0.5 * gh) - (y1 + 0.5 * eh)) / eh * 10.0
    dw = jnp.log(gw / ew) * 5.0
    dh = jnp.log(gh / eh) * 5.0

    fg = lab > 0
    zero = jnp.zeros_like(dx)
    wv = jnp.where(fg, zero + 1.0, zero)
    tgt_ref[0, 0, :] = jnp.where(fg, dx, zero)
    tgt_ref[0, 1, :] = jnp.where(fg, dy, zero)
    tgt_ref[0, 2, :] = jnp.where(fg, dw, zero)
    tgt_ref[0, 3, :] = jnp.where(fg, dh, zero)
    w_ref[0, 0, :] = wv
    w_ref[0, 1, :] = wv
    w_ref[0, 2, :] = wv
    w_ref[0, 3, :] = wv
    w2_ref[0, 0, :] = wv
    w2_ref[0, 1, :] = wv
    w2_ref[0, 2, :] = wv
    w2_ref[0, 3, :] = wv


_out_bs = pl.BlockSpec((1, 4, BLK), lambda i: (0, 0, i))

_tc_call = pl.pallas_call(
    _tc_body,
    grid=(GRID,),
    in_specs=[
        pl.BlockSpec((5, 1, BLK), lambda i: (0, 0, i)),
        pl.BlockSpec((1, 4, BLK), lambda i: (0, 0, i)),
        pl.BlockSpec((BLK,), lambda i: (i,)),
    ],
    out_specs=[_out_bs, _out_bs, _out_bs],
    out_shape=(
        jax.ShapeDtypeStruct((1, 4, N), jnp.float32),
        jax.ShapeDtypeStruct((1, 4, N), jnp.float32),
        jax.ShapeDtypeStruct((1, 4, N), jnp.float32),
    ),
)


@jax.jit
def kernel(gt_rois, rois, labels):
    rois_t = jnp.transpose(rois, (2, 0, 1))
    gt_t = jnp.transpose(gt_rois, (0, 2, 1))
    tgt_t, w_t, w2_t = _tc_call(rois_t, gt_t, labels)
    tgt = jnp.transpose(tgt_t, (0, 2, 1))
    w = jnp.transpose(w_t, (0, 2, 1))
    w2 = jnp.transpose(w2_t, (0, 2, 1))
    return tgt, w, w2
